# baseline (speedup 1.0000x reference)
import functools

import numpy as np
import jax
import jax.numpy as jnp
from jax import lax
from jax.experimental import pallas as pl
from jax.experimental.pallas import tpu as pltpu
from jax.experimental.pallas import tpu_sc as plsc

N_NODES = 10000
N_EDGES = 40000
MUL_S = 48
MUL_V = 16
NODE_DIM = 96
SH_DIM = 4
N_EDGE_FEAT = 128
HIDDEN = 128
C_PATH = 0.125
EPS = 1e-5

NC = 2
NS = 16
NW = NC * NS
E_PAD = 40960
EPW = E_PAD // NW
CHUNK = 128
NCH = EPW // CHUNK
N_ACC = 10240
NPW = N_ACC // NS

BE = 1024
BN = 512


def _make_gather_body(epw, nch):
    def body(table_hbm, idx_hbm, out_hbm, idx_v, rows_v,
             sem0, sem1, sem2, sem3):
        wid = lax.axis_index("s") * NC + lax.axis_index("c")
        sems = (sem0, sem1, sem2, sem3)
        pltpu.sync_copy(idx_hbm.at[wid], idx_v)
        cps = [None] * nch
        for j in range(min(3, nch)):
            cps[j] = pltpu.async_copy(table_hbm.at[idx_v.at[j]],
                                      rows_v.at[j], sems[j])
        for j in range(nch):
            cps[j].wait()
            if j + 3 < nch:
                cps[j + 3] = pltpu.async_copy(table_hbm.at[idx_v.at[j + 3]],
                                              rows_v.at[(j + 3) % 4],
                                              sems[(j + 3) % 4])
            pltpu.sync_copy(rows_v.at[j % 4],
                            out_hbm.at[pl.ds(wid * epw + j * CHUNK, CHUNK)])
    return body


def _sc_gather(table_f32, idx3d, n_edges):
    epw = n_edges // NW
    nch = epw // CHUNK
    fn = pl.kernel(
        _make_gather_body(epw, nch),
        out_type=jax.ShapeDtypeStruct((n_edges, 128), jnp.float32),
        mesh=plsc.VectorSubcoreMesh(
            core_axis_name="c", subcore_axis_name="s", num_cores=NC,
            num_subcores=NS,
        ),
        scratch_types=[
            pltpu.VMEM((nch, CHUNK), jnp.int32),
            pltpu.VMEM((4, CHUNK, 128), jnp.float32),
            pltpu.SemaphoreType.DMA,
            pltpu.SemaphoreType.DMA,
            pltpu.SemaphoreType.DMA,
            pltpu.SemaphoreType.DMA,
        ],
    )
    return fn(table_f32, idx3d)


def _make_scatter_body(epw, nch):
    def body(tp_hbm, idx_hbm, z_hbm, out_hbm, idx_v, rows_v, acc_sp, lsem):
        c = lax.axis_index("c")
        s = lax.axis_index("s")
        wid = s * NC + c
        pltpu.sync_copy(z_hbm, acc_sp.at[pl.ds(s * NPW, NPW)])
        pltpu.sync_copy(idx_hbm.at[wid], idx_v)
        loads = [None] * nch
        loads[0] = pltpu.async_copy(tp_hbm.at[pl.ds(wid * epw, CHUNK)],
                                    rows_v.at[0], lsem)
        plsc.subcore_barrier()
        for j in range(nch):
            loads[j].wait()
            if j + 1 < nch:
                loads[j + 1] = pltpu.async_copy(
                    tp_hbm.at[pl.ds(wid * epw + (j + 1) * CHUNK, CHUNK)],
                    rows_v.at[(j + 1) % 2], lsem)
            pltpu.sync_copy(rows_v.at[j % 2], acc_sp.at[idx_v.at[j]],
                            add=True)
        plsc.subcore_barrier()
        base = c * N_ACC + s * NPW
        pltpu.sync_copy(acc_sp.at[pl.ds(s * NPW, NPW)],
                        out_hbm.at[pl.ds(base, NPW)])
    return body


def _sc_scatter(tp, idx3d, z128, n_edges):
    epw = n_edges // NW
    nch = epw // CHUNK
    fn = pl.kernel(
        _make_scatter_body(epw, nch),
        out_type=jax.ShapeDtypeStruct((NC * N_ACC, 128), jnp.float32),
        mesh=plsc.VectorSubcoreMesh(
            core_axis_name="c", subcore_axis_name="s", num_cores=NC,
            num_subcores=NS,
        ),
        scratch_types=[
            pltpu.VMEM((nch, CHUNK), jnp.int32),
            pltpu.VMEM((2, CHUNK, 128), jnp.float32),
            pltpu.VMEM_SHARED((N_ACC, 128), jnp.float32),
            pltpu.SemaphoreType.DMA,
        ],
    )
    return fn(tp, idx3d, z128)


def _fold_to(p, target):
    w = p.shape[1]
    while w > target:
        w //= 2
        p = p[:, :w] + p[:, w:2 * w]
    return p


def _tc_edge_body(ea_ref, sh_ref, xd_ref, w1_ref, b1_ref, gs_ref,
                  gv_ref, rs_ref, rv_ref, sel_ref,
                  q3_ref, sf_ref, bm_ref, o_ref):
    f32 = jnp.float32
    bf16 = jnp.bfloat16
    ea = ea_ref[...]
    sh = sh_ref[...]
    xd = xd_ref[:, :NODE_DIM]
    h = jax.nn.relu(
        jnp.dot(ea, w1_ref[...], preferred_element_type=f32)
        + b1_ref[0:1, :])
    hb = h.astype(bf16)
    ws = jnp.dot(hb, gs_ref[...], preferred_element_type=f32)
    wv = jnp.dot(hb, gv_ref[...], preferred_element_type=f32)
    xs = xd[:, :MUL_S]
    xvf = xd[:, MUL_S:]
    shs = sh[:, 0:1]
    shv = sh[:, 1:4]
    vv = jnp.concatenate([shv] * MUL_V, axis=1)
    bb = jnp.dot(xvf * vv, sel_ref[...])
    a_s = jnp.concatenate([xs * shs, bb], axis=1).astype(bf16)
    u_s = jnp.dot(a_s, rs_ref[...], preferred_element_type=f32)
    parts = [_fold_to(ws * u_s, 384)]
    avs = [a_s]
    xvp = jnp.dot(xvf, q3_ref[...],
                  preferred_element_type=f32)
    for j in range(3):
        a_vj = jnp.concatenate(
            [xs * shv[:, j:j + 1],
             xvp[:, j * MUL_V:(j + 1) * MUL_V] * shs],
            axis=1).astype(bf16)
        avs.append(a_vj)
        u_vj = jnp.dot(a_vj, rv_ref[...], preferred_element_type=f32)
        parts.append(_fold_to(wv * u_vj, 128))
    big = jnp.concatenate(parts, axis=1).astype(bf16)
    abig = jnp.concatenate(avs, axis=1)
    out = C_PATH * (jnp.dot(big, sf_ref[...], preferred_element_type=f32)
                    + jnp.dot(abig, bm_ref[...],
                              preferred_element_type=f32))
    n = out.shape[0]
    pad = jnp.concatenate(
        [jnp.ones((n, 1), jnp.float32), jnp.zeros((n, 31), jnp.float32)],
        axis=1)
    o_ref[...] = jnp.concatenate([out, pad], axis=1)


def _tc_edge(ea, sh, xd, w1, b1, gs, gv, rs, rv, sel, q3, sf, bm):
    n_blk = ea.shape[0] // BE
    full = lambda r, c: pl.BlockSpec((r, c), lambda i: (0, 0))
    out = pl.pallas_call(
        _tc_edge_body,
        grid=(n_blk,),
        in_specs=[
            pl.BlockSpec((BE, N_EDGE_FEAT), lambda i: (i, 0)),
            pl.BlockSpec((BE, SH_DIM), lambda i: (i, 0)),
            pl.BlockSpec((BE, 128), lambda i: (i, 0)),
            full(N_EDGE_FEAT, HIDDEN),
            full(8, HIDDEN),
            full(HIDDEN, 3072),
            full(HIDDEN, 1024),
            full(64, 3072),
            full(64, 1024),
            full(MUL_S, MUL_V),
            full(MUL_S, MUL_S),
            full(768, NODE_DIM),
            full(256, NODE_DIM),
        ],
        out_specs=pl.BlockSpec((BE, 128), lambda i: (i, 0)),
        out_shape=jax.ShapeDtypeStruct((ea.shape[0], 128), jnp.float32),
    )
    return out(ea, sh, xd, w1, b1, gs, gv, rs, rv, sel, q3, sf, bm)


def _tc_ln_body(p0_ref, p1_ref, na_ref, lnc_ref, m2_ref, o_ref):
    psum = p0_ref[...] + p1_ref[...]
    ssum = psum[:, :NODE_DIM]
    cnt = psum[:, NODE_DIM:NODE_DIM + 1]
    x = ssum / jnp.maximum(cnt, 1.0) + na_ref[...]
    lnc = lnc_ref[...]
    w_s = lnc[0:1, :]
    b_s = lnc[1:2, :]
    ms_s = lnc[2:3, :]
    w_v = lnc[3:4, :]
    ms_v = lnc[4:5, :]
    f1 = x[:, :MUL_S]
    m1 = jnp.mean(f1, axis=1, keepdims=True)
    f1 = f1 - m1 * ms_s
    n1 = jnp.mean(f1 * f1, axis=1, keepdims=True)
    f1 = f1 * (lax.rsqrt(n1 + EPS) * w_s) + b_s
    x2 = x[:, MUL_S:]
    m2f = jnp.dot(x2, m2_ref[...])
    f2 = x2 - m2f * ms_v
    n2 = jnp.mean(f2 * f2, axis=1, keepdims=True)
    f2 = f2 * (lax.rsqrt(n2 + EPS) * w_v)
    o_ref[...] = jnp.concatenate([f1, f2], axis=1)


def _tc_ln(p0, p1, na_pad, lnc, m2c):
    n_blk = N_ACC // BN
    out = pl.pallas_call(
        _tc_ln_body,
        grid=(n_blk,),
        in_specs=[
            pl.BlockSpec((BN, 128), lambda i: (i, 0)),
            pl.BlockSpec((BN, 128), lambda i: (i, 0)),
            pl.BlockSpec((BN, NODE_DIM), lambda i: (i, 0)),
            pl.BlockSpec((8, MUL_S), lambda i: (0, 0)),
            pl.BlockSpec((MUL_S, MUL_S), lambda i: (0, 0)),
        ],
        out_specs=pl.BlockSpec((BN, NODE_DIM), lambda i: (i, 0)),
        out_shape=jax.ShapeDtypeStruct((N_ACC, NODE_DIM), jnp.float32),
    )
    return out(p0, p1, na_pad, lnc, m2c)


def _pad8(row):
    return np.pad(row[None, :], ((0, 7), (0, 0))).astype(np.float32)


_R_S = np.kron(np.eye(64), np.ones((1, MUL_S))).astype(np.float32)
_S_S = np.kron(np.ones((64, 1)), np.eye(MUL_S)).astype(np.float32)
_R_V = np.kron(np.eye(64), np.ones((1, MUL_V))).astype(np.float32)
_S_V = np.kron(np.ones((64, 1)), np.eye(MUL_V)).astype(np.float32)
_SEL = np.kron(np.eye(MUL_V), np.ones((3, 1))).astype(np.float32)
_Q3 = np.zeros((MUL_S, MUL_S), np.float32)
for _u in range(MUL_V):
    for _j in range(3):
        _Q3[3 * _u + _j, _j * MUL_V + _u] = 1.0
_SF = np.zeros((768, NODE_DIM), np.float32)
_SF[:384, :MUL_S] = np.kron(np.ones((8, 1)), np.eye(MUL_S))
for _j in range(3):
    _pj = np.zeros((MUL_V, MUL_S), np.float32)
    for _w in range(MUL_V):
        _pj[_w, 3 * _w + _j] = 1.0
    _SF[384 + _j * 128:384 + (_j + 1) * 128, MUL_S:] = np.kron(
        np.ones((8, 1)), _pj)
_M2 = np.kron(np.ones((MUL_V, MUL_V), np.float32) / MUL_V,
              np.eye(3, dtype=np.float32))


def kernel(node_attr, edge_index, edge_attr, edge_sh, fc_w1, fc_b1, fc_w2,
           fc_b2, ln_weight, ln_bias, ln_mean_shift):
    f32 = jnp.float32
    edge_dst = jnp.concatenate(
        [edge_index[1], jnp.zeros((E_PAD - N_EDGES,), jnp.int32)])
    edge_src = jnp.concatenate(
        [edge_index[0], jnp.full((E_PAD - N_EDGES,), N_NODES, jnp.int32)])
    ea_pad = jnp.concatenate(
        [edge_attr, jnp.zeros((E_PAD - N_EDGES, N_EDGE_FEAT), f32)])
    sh_pad = jnp.concatenate(
        [edge_sh, jnp.zeros((E_PAD - N_EDGES, SH_DIM), f32)])
    na_pad = jnp.concatenate(
        [node_attr, jnp.zeros((N_ACC - N_NODES, NODE_DIM), f32)])
    na128 = jnp.concatenate(
        [node_attr, jnp.zeros((N_NODES, 128 - NODE_DIM), f32)], axis=1)

    bf16 = jnp.bfloat16
    g_s = jnp.concatenate([fc_w2[:, :2304], fc_w2[:, 3328:]],
                          axis=1).astype(bf16)
    g_v = fc_w2[:, 2304:3328].astype(bf16)
    b1r = jnp.tile(fc_b1[None, :], (8, 1))
    b2s = jnp.concatenate([fc_b2[:2304], fc_b2[3328:]]).reshape(64, MUL_S)
    b2v = fc_b2[2304:3328].reshape(64, MUL_V)
    bmat = jnp.zeros((256, NODE_DIM), f32)
    bmat = bmat.at[:64, :MUL_S].set(b2s)
    for _j in range(3):
        bmat = bmat.at[64 * (_j + 1):64 * (_j + 2),
                       MUL_S + _j::3].set(b2v)
    bmat = bmat.astype(bf16)

    lnc = jnp.zeros((8, MUL_S), f32)
    lnc = lnc.at[0].set(ln_weight[:MUL_S])
    lnc = lnc.at[1].set(ln_bias)
    lnc = lnc.at[2].set(ln_mean_shift[0, :MUL_S, 0])
    lnc = lnc.at[3].set(jnp.repeat(ln_weight[MUL_S:], 3))
    lnc = lnc.at[4].set(jnp.repeat(ln_mean_shift[0, MUL_S:, 0], 3))

    z128 = jnp.zeros((NPW, 128), f32)

    dst3d = edge_dst.reshape(NW, NCH, CHUNK)
    src3d = edge_src.reshape(NW, NCH, CHUNK)
    consts = (fc_w1.astype(bf16), b1r, g_s, g_v,
              jnp.asarray(_R_S, bf16), jnp.asarray(_R_V, bf16),
              jnp.asarray(_SEL), jnp.asarray(_Q3), jnp.asarray(_SF, bf16),
              bmat)
    x_dst = _sc_gather(na128, dst3d, E_PAD)
    tp = _tc_edge(ea_pad.astype(bf16), sh_pad, x_dst, *consts)
    summed = _sc_scatter(tp, src3d, z128, E_PAD).reshape(NC, N_ACC, 128)
    out = _tc_ln(summed[0], summed[1], na_pad, lnc, jnp.asarray(_M2))
    return out[:N_NODES]


# speedup vs baseline: 1.3223x; 1.3223x over previous
import functools

import numpy as np
import jax
import jax.numpy as jnp
from jax import lax
from jax.experimental import pallas as pl
from jax.experimental.pallas import tpu as pltpu
from jax.experimental.pallas import tpu_sc as plsc

N_NODES = 10000
N_EDGES = 40000
MUL_S = 48
MUL_V = 16
NODE_DIM = 96
SH_DIM = 4
N_EDGE_FEAT = 128
HIDDEN = 128
C_PATH = 0.125
EPS = 1e-5

NC = 2
NS = 16
NW = NC * NS
E_PAD = 40960
EPW = E_PAD // NW
CHUNK = 128
NCH = EPW // CHUNK
N_ACC = 10240
NPW = N_ACC // NS

BE = 1024
BN = 512


def _make_gather_body(epw, nch):
    def body(table_hbm, idx_hbm, out_hbm, idx_v, rows_v,
             sem0, sem1, sem2, sem3):
        wid = lax.axis_index("s") * NC + lax.axis_index("c")
        sems = (sem0, sem1, sem2, sem3)
        pltpu.sync_copy(idx_hbm.at[wid], idx_v)
        cps = [None] * nch
        for j in range(min(3, nch)):
            cps[j] = pltpu.async_copy(table_hbm.at[idx_v.at[j]],
                                      rows_v.at[j], sems[j])
        for j in range(nch):
            cps[j].wait()
            if j + 3 < nch:
                cps[j + 3] = pltpu.async_copy(table_hbm.at[idx_v.at[j + 3]],
                                              rows_v.at[(j + 3) % 4],
                                              sems[(j + 3) % 4])
            pltpu.sync_copy(rows_v.at[j % 4],
                            out_hbm.at[pl.ds(wid * epw + j * CHUNK, CHUNK)])
    return body


def _sc_gather(table_f32, idx3d, n_edges):
    epw = n_edges // NW
    nch = epw // CHUNK
    fn = pl.kernel(
        _make_gather_body(epw, nch),
        out_type=jax.ShapeDtypeStruct((n_edges, 128), jnp.float32),
        mesh=plsc.VectorSubcoreMesh(
            core_axis_name="c", subcore_axis_name="s", num_cores=NC,
            num_subcores=NS,
        ),
        scratch_types=[
            pltpu.VMEM((nch, CHUNK), jnp.int32),
            pltpu.VMEM((4, CHUNK, 128), jnp.float32),
            pltpu.SemaphoreType.DMA,
            pltpu.SemaphoreType.DMA,
            pltpu.SemaphoreType.DMA,
            pltpu.SemaphoreType.DMA,
        ],
    )
    return fn(table_f32, idx3d)


def _make_scatter_body(epw, nch):
    def body(tp_hbm, idx_hbm, z_hbm, out_hbm, idx_v, rows_v, acc_sp, lsem):
        c = lax.axis_index("c")
        s = lax.axis_index("s")
        wid = s * NC + c
        pltpu.sync_copy(z_hbm, acc_sp.at[pl.ds(s * NPW, NPW)])
        pltpu.sync_copy(idx_hbm.at[wid], idx_v)
        loads = [None] * nch
        loads[0] = pltpu.async_copy(tp_hbm.at[pl.ds(wid * epw, CHUNK)],
                                    rows_v.at[0], lsem)
        plsc.subcore_barrier()
        for j in range(nch):
            loads[j].wait()
            if j + 1 < nch:
                loads[j + 1] = pltpu.async_copy(
                    tp_hbm.at[pl.ds(wid * epw + (j + 1) * CHUNK, CHUNK)],
                    rows_v.at[(j + 1) % 2], lsem)
            pltpu.sync_copy(rows_v.at[j % 2], acc_sp.at[idx_v.at[j]],
                            add=True)
        plsc.subcore_barrier()
        base = c * N_ACC + s * NPW
        pltpu.sync_copy(acc_sp.at[pl.ds(s * NPW, NPW)],
                        out_hbm.at[pl.ds(base, NPW)])
    return body


def _sc_scatter(tp, idx3d, z128, n_edges):
    epw = n_edges // NW
    nch = epw // CHUNK
    fn = pl.kernel(
        _make_scatter_body(epw, nch),
        out_type=jax.ShapeDtypeStruct((NC * N_ACC, 128), jnp.float32),
        mesh=plsc.VectorSubcoreMesh(
            core_axis_name="c", subcore_axis_name="s", num_cores=NC,
            num_subcores=NS,
        ),
        scratch_types=[
            pltpu.VMEM((nch, CHUNK), jnp.int32),
            pltpu.VMEM((2, CHUNK, 128), jnp.float32),
            pltpu.VMEM_SHARED((N_ACC, 128), jnp.float32),
            pltpu.SemaphoreType.DMA,
        ],
    )
    return fn(tp, idx3d, z128)


def _fold_to(p, target):
    w = p.shape[1]
    while w > target:
        w //= 2
        p = p[:, :w] + p[:, w:2 * w]
    return p


def _tc_edge_body(ea_ref, sh_ref, xd_ref, w1_ref, b1_ref, gs_ref,
                  gv_ref, rs_ref, rv_ref, sel_ref,
                  q3_ref, sf_ref, bm_ref, o_ref):
    f32 = jnp.float32
    bf16 = jnp.bfloat16
    ea = ea_ref[...]
    sh = sh_ref[...]
    xd = xd_ref[:, :NODE_DIM]
    h = jax.nn.relu(
        jnp.dot(ea, w1_ref[...], preferred_element_type=f32)
        + b1_ref[0:1, :])
    hb = h.astype(bf16)
    ws = jnp.dot(hb, gs_ref[...], preferred_element_type=f32)
    wv = jnp.dot(hb, gv_ref[...], preferred_element_type=f32)
    xs = xd[:, :MUL_S]
    xvf = xd[:, MUL_S:]
    shs = sh[:, 0:1]
    shv = sh[:, 1:4]
    vv = jnp.concatenate([shv] * MUL_V, axis=1)
    bb = jnp.dot(xvf * vv, sel_ref[...])
    a_s = jnp.concatenate([xs * shs, bb], axis=1).astype(bf16)
    u_s = jnp.dot(a_s, rs_ref[...], preferred_element_type=f32)
    parts = [_fold_to(ws * u_s, 384)]
    avs = [a_s]
    xvp = jnp.dot(xvf, q3_ref[...],
                  preferred_element_type=f32)
    for j in range(3):
        a_vj = jnp.concatenate(
            [xs * shv[:, j:j + 1],
             xvp[:, j * MUL_V:(j + 1) * MUL_V] * shs],
            axis=1).astype(bf16)
        avs.append(a_vj)
        u_vj = jnp.dot(a_vj, rv_ref[...], preferred_element_type=f32)
        parts.append(_fold_to(wv * u_vj, 128))
    big = jnp.concatenate(parts, axis=1).astype(bf16)
    abig = jnp.concatenate(avs, axis=1)
    out = C_PATH * (jnp.dot(big, sf_ref[...], preferred_element_type=f32)
                    + jnp.dot(abig, bm_ref[...],
                              preferred_element_type=f32))
    n = out.shape[0]
    pad = jnp.concatenate(
        [jnp.ones((n, 1), jnp.float32), jnp.zeros((n, 31), jnp.float32)],
        axis=1)
    o_ref[...] = jnp.concatenate([out, pad], axis=1)


def _tc_edge(ea, sh, xd, w1, b1, gs, gv, rs, rv, sel, q3, sf, bm):
    n_blk = ea.shape[0] // BE
    full = lambda r, c: pl.BlockSpec((r, c), lambda i: (0, 0))
    out = pl.pallas_call(
        _tc_edge_body,
        grid=(n_blk,),
        in_specs=[
            pl.BlockSpec((BE, N_EDGE_FEAT), lambda i: (i, 0)),
            pl.BlockSpec((BE, SH_DIM), lambda i: (i, 0)),
            pl.BlockSpec((BE, 128), lambda i: (i, 0)),
            full(N_EDGE_FEAT, HIDDEN),
            full(8, HIDDEN),
            full(HIDDEN, 3072),
            full(HIDDEN, 1024),
            full(64, 3072),
            full(64, 1024),
            full(MUL_S, MUL_V),
            full(MUL_S, MUL_S),
            full(768, NODE_DIM),
            full(256, NODE_DIM),
        ],
        out_specs=pl.BlockSpec((BE, 128), lambda i: (i, 0)),
        out_shape=jax.ShapeDtypeStruct((ea.shape[0], 128), jnp.float32),
    )
    return out(ea, sh, xd, w1, b1, gs, gv, rs, rv, sel, q3, sf, bm)


def _tc_ln_body(p0_ref, p1_ref, na_ref, lnc_ref, m2_ref, o_ref):
    psum = p0_ref[...] + p1_ref[...]
    ssum = psum[:, :NODE_DIM]
    cnt = psum[:, NODE_DIM:NODE_DIM + 1]
    x = ssum / jnp.maximum(cnt, 1.0) + na_ref[...]
    lnc = lnc_ref[...]
    w_s = lnc[0:1, :]
    b_s = lnc[1:2, :]
    ms_s = lnc[2:3, :]
    w_v = lnc[3:4, :]
    ms_v = lnc[4:5, :]
    f1 = x[:, :MUL_S]
    m1 = jnp.mean(f1, axis=1, keepdims=True)
    f1 = f1 - m1 * ms_s
    n1 = jnp.mean(f1 * f1, axis=1, keepdims=True)
    f1 = f1 * (lax.rsqrt(n1 + EPS) * w_s) + b_s
    x2 = x[:, MUL_S:]
    m2f = jnp.dot(x2, m2_ref[...])
    f2 = x2 - m2f * ms_v
    n2 = jnp.mean(f2 * f2, axis=1, keepdims=True)
    f2 = f2 * (lax.rsqrt(n2 + EPS) * w_v)
    o_ref[...] = jnp.concatenate([f1, f2], axis=1)


def _tc_ln(p0, p1, na_pad, lnc, m2c):
    n_blk = N_ACC // BN
    out = pl.pallas_call(
        _tc_ln_body,
        grid=(n_blk,),
        in_specs=[
            pl.BlockSpec((BN, 128), lambda i: (i, 0)),
            pl.BlockSpec((BN, 128), lambda i: (i, 0)),
            pl.BlockSpec((BN, NODE_DIM), lambda i: (i, 0)),
            pl.BlockSpec((8, MUL_S), lambda i: (0, 0)),
            pl.BlockSpec((MUL_S, MUL_S), lambda i: (0, 0)),
        ],
        out_specs=pl.BlockSpec((BN, NODE_DIM), lambda i: (i, 0)),
        out_shape=jax.ShapeDtypeStruct((N_ACC, NODE_DIM), jnp.float32),
    )
    return out(p0, p1, na_pad, lnc, m2c)


def _pad8(row):
    return np.pad(row[None, :], ((0, 7), (0, 0))).astype(np.float32)


_R_S = np.kron(np.eye(64), np.ones((1, MUL_S))).astype(np.float32)
_S_S = np.kron(np.ones((64, 1)), np.eye(MUL_S)).astype(np.float32)
_R_V = np.kron(np.eye(64), np.ones((1, MUL_V))).astype(np.float32)
_S_V = np.kron(np.ones((64, 1)), np.eye(MUL_V)).astype(np.float32)
_SEL = np.kron(np.eye(MUL_V), np.ones((3, 1))).astype(np.float32)
_Q3 = np.zeros((MUL_S, MUL_S), np.float32)
for _u in range(MUL_V):
    for _j in range(3):
        _Q3[3 * _u + _j, _j * MUL_V + _u] = 1.0
_SF = np.zeros((768, NODE_DIM), np.float32)
_SF[:384, :MUL_S] = np.kron(np.ones((8, 1)), np.eye(MUL_S))
for _j in range(3):
    _pj = np.zeros((MUL_V, MUL_S), np.float32)
    for _w in range(MUL_V):
        _pj[_w, 3 * _w + _j] = 1.0
    _SF[384 + _j * 128:384 + (_j + 1) * 128, MUL_S:] = np.kron(
        np.ones((8, 1)), _pj)
_M2 = np.kron(np.ones((MUL_V, MUL_V), np.float32) / MUL_V,
              np.eye(3, dtype=np.float32))


def kernel(node_attr, edge_index, edge_attr, edge_sh, fc_w1, fc_b1, fc_w2,
           fc_b2, ln_weight, ln_bias, ln_mean_shift):
    f32 = jnp.float32
    edge_dst = jnp.concatenate(
        [edge_index[1], jnp.zeros((E_PAD - N_EDGES,), jnp.int32)])
    edge_src = jnp.concatenate(
        [edge_index[0], jnp.full((E_PAD - N_EDGES,), N_NODES, jnp.int32)])
    ea_pad = jnp.concatenate(
        [edge_attr, jnp.zeros((E_PAD - N_EDGES, N_EDGE_FEAT), f32)])
    sh_pad = jnp.concatenate(
        [edge_sh, jnp.zeros((E_PAD - N_EDGES, SH_DIM), f32)])
    na_pad = jnp.concatenate(
        [node_attr, jnp.zeros((N_ACC - N_NODES, NODE_DIM), f32)])
    na128 = jnp.concatenate(
        [node_attr, jnp.zeros((N_NODES, 128 - NODE_DIM), f32)], axis=1)

    bf16 = jnp.bfloat16
    g_s = jnp.concatenate([fc_w2[:, :2304], fc_w2[:, 3328:]],
                          axis=1).astype(bf16)
    g_v = fc_w2[:, 2304:3328].astype(bf16)
    b1r = jnp.tile(fc_b1[None, :], (8, 1))
    b2s = jnp.concatenate([fc_b2[:2304], fc_b2[3328:]]).reshape(64, MUL_S)
    b2v = fc_b2[2304:3328].reshape(64, MUL_V)
    eye3 = jnp.eye(3, dtype=f32)
    zb = jnp.zeros((64, MUL_S), f32)
    blocks = [jnp.concatenate([b2s, zb], axis=1)]
    for _j in range(3):
        pv = (b2v[:, :, None] * eye3[_j][None, None, :]).reshape(64, MUL_S)
        blocks.append(jnp.concatenate([zb, pv], axis=1))
    bmat = jnp.concatenate(blocks, axis=0).astype(jnp.bfloat16)

    lnc = jnp.zeros((8, MUL_S), f32)
    lnc = lnc.at[0].set(ln_weight[:MUL_S])
    lnc = lnc.at[1].set(ln_bias)
    lnc = lnc.at[2].set(ln_mean_shift[0, :MUL_S, 0])
    lnc = lnc.at[3].set(jnp.repeat(ln_weight[MUL_S:], 3))
    lnc = lnc.at[4].set(jnp.repeat(ln_mean_shift[0, MUL_S:, 0], 3))

    z128 = jnp.zeros((NPW, 128), f32)

    dst3d = edge_dst.reshape(NW, NCH, CHUNK)
    src3d = edge_src.reshape(NW, NCH, CHUNK)
    consts = (fc_w1.astype(bf16), b1r, g_s, g_v,
              jnp.asarray(_R_S, bf16), jnp.asarray(_R_V, bf16),
              jnp.asarray(_SEL), jnp.asarray(_Q3), jnp.asarray(_SF, bf16),
              bmat)
    x_dst = _sc_gather(na128, dst3d, E_PAD)
    tp = _tc_edge(ea_pad.astype(bf16), sh_pad, x_dst, *consts)
    summed = _sc_scatter(tp, src3d, z128, E_PAD).reshape(NC, N_ACC, 128)
    out = _tc_ln(summed[0], summed[1], na_pad, lnc, jnp.asarray(_M2))
    return out[:N_NODES]


# speedup vs baseline: 1.3230x; 1.0005x over previous
import functools

import numpy as np
import jax
import jax.numpy as jnp
from jax import lax
from jax.experimental import pallas as pl
from jax.experimental.pallas import tpu as pltpu
from jax.experimental.pallas import tpu_sc as plsc

N_NODES = 10000
N_EDGES = 40000
MUL_S = 48
MUL_V = 16
NODE_DIM = 96
SH_DIM = 4
N_EDGE_FEAT = 128
HIDDEN = 128
C_PATH = 0.125
EPS = 1e-5

NC = 2
NS = 16
NW = NC * NS
E_PAD = 40960
EPW = E_PAD // NW
CHUNK = 128
NCH = EPW // CHUNK
N_ACC = 10240
NPW = N_ACC // NS

BE = 1024
BN = 512


def _make_gather_body(epw, nch):
    def body(table_hbm, idx_hbm, out_hbm, idx_v, rows_v,
             sem0, sem1, sem2, sem3, wsem0, wsem1):
        wid = lax.axis_index("s") * NC + lax.axis_index("c")
        sems = (sem0, sem1, sem2, sem3)
        wsems = (wsem0, wsem1)
        pltpu.sync_copy(idx_hbm.at[wid], idx_v)
        cps = [None] * nch
        wcps = [None] * nch
        for j in range(min(3, nch)):
            cps[j] = pltpu.async_copy(table_hbm.at[idx_v.at[j]],
                                      rows_v.at[j], sems[j])
        for j in range(nch):
            cps[j].wait()
            if j + 3 < nch:
                if j >= 1 and wcps[j - 1] is not None:
                    wcps[j - 1].wait()
                    wcps[j - 1] = None
                cps[j + 3] = pltpu.async_copy(table_hbm.at[idx_v.at[j + 3]],
                                              rows_v.at[(j + 3) % 4],
                                              sems[(j + 3) % 4])
            wcps[j] = pltpu.async_copy(
                rows_v.at[j % 4],
                out_hbm.at[pl.ds(wid * epw + j * CHUNK, CHUNK)],
                wsems[j % 2])
        for j in range(nch):
            if wcps[j] is not None:
                wcps[j].wait()
    return body


def _sc_gather(table_f32, idx3d, n_edges):
    epw = n_edges // NW
    nch = epw // CHUNK
    fn = pl.kernel(
        _make_gather_body(epw, nch),
        out_type=jax.ShapeDtypeStruct((n_edges, 128), jnp.float32),
        mesh=plsc.VectorSubcoreMesh(
            core_axis_name="c", subcore_axis_name="s", num_cores=NC,
            num_subcores=NS,
        ),
        scratch_types=[
            pltpu.VMEM((nch, CHUNK), jnp.int32),
            pltpu.VMEM((4, CHUNK, 128), jnp.float32),
            pltpu.SemaphoreType.DMA,
            pltpu.SemaphoreType.DMA,
            pltpu.SemaphoreType.DMA,
            pltpu.SemaphoreType.DMA,
            pltpu.SemaphoreType.DMA,
            pltpu.SemaphoreType.DMA,
        ],
    )
    return fn(table_f32, idx3d)


def _make_scatter_body(epw, nch):
    def body(tp_hbm, idx_hbm, z_hbm, out_hbm, idx_v, rows_v, acc_sp, lsem):
        c = lax.axis_index("c")
        s = lax.axis_index("s")
        wid = s * NC + c
        pltpu.sync_copy(z_hbm, acc_sp.at[pl.ds(s * NPW, NPW)])
        pltpu.sync_copy(idx_hbm.at[wid], idx_v)
        loads = [None] * nch
        loads[0] = pltpu.async_copy(tp_hbm.at[pl.ds(wid * epw, CHUNK)],
                                    rows_v.at[0], lsem)
        plsc.subcore_barrier()
        for j in range(nch):
            loads[j].wait()
            if j + 1 < nch:
                loads[j + 1] = pltpu.async_copy(
                    tp_hbm.at[pl.ds(wid * epw + (j + 1) * CHUNK, CHUNK)],
                    rows_v.at[(j + 1) % 2], lsem)
            pltpu.sync_copy(rows_v.at[j % 2], acc_sp.at[idx_v.at[j]],
                            add=True)
        plsc.subcore_barrier()
        base = c * N_ACC + s * NPW
        pltpu.sync_copy(acc_sp.at[pl.ds(s * NPW, NPW)],
                        out_hbm.at[pl.ds(base, NPW)])
    return body


def _sc_scatter(tp, idx3d, z128, n_edges):
    epw = n_edges // NW
    nch = epw // CHUNK
    fn = pl.kernel(
        _make_scatter_body(epw, nch),
        out_type=jax.ShapeDtypeStruct((NC * N_ACC, 128), jnp.float32),
        mesh=plsc.VectorSubcoreMesh(
            core_axis_name="c", subcore_axis_name="s", num_cores=NC,
            num_subcores=NS,
        ),
        scratch_types=[
            pltpu.VMEM((nch, CHUNK), jnp.int32),
            pltpu.VMEM((2, CHUNK, 128), jnp.float32),
            pltpu.VMEM_SHARED((N_ACC, 128), jnp.float32),
            pltpu.SemaphoreType.DMA,
        ],
    )
    return fn(tp, idx3d, z128)


def _fold_to(p, target):
    w = p.shape[1]
    while w > target:
        w //= 2
        p = p[:, :w] + p[:, w:2 * w]
    return p


def _tc_edge_body(ea_ref, sh_ref, xd_ref, w1_ref, b1_ref, gs_ref,
                  gv_ref, rs_ref, rv_ref, sel_ref,
                  q3_ref, sf_ref, bm_ref, o_ref):
    f32 = jnp.float32
    bf16 = jnp.bfloat16
    ea = ea_ref[...]
    sh = sh_ref[...]
    xd = xd_ref[:, :NODE_DIM]
    h = jax.nn.relu(
        jnp.dot(ea, w1_ref[...], preferred_element_type=f32)
        + b1_ref[0:1, :])
    hb = h.astype(bf16)
    ws = jnp.dot(hb, gs_ref[...], preferred_element_type=f32)
    wv = jnp.dot(hb, gv_ref[...], preferred_element_type=f32)
    xs = xd[:, :MUL_S]
    xvf = xd[:, MUL_S:]
    shs = sh[:, 0:1]
    shv = sh[:, 1:4]
    vv = jnp.concatenate([shv] * MUL_V, axis=1)
    bb = jnp.dot(xvf * vv, sel_ref[...])
    a_s = jnp.concatenate([xs * shs, bb], axis=1).astype(bf16)
    u_s = jnp.dot(a_s, rs_ref[...], preferred_element_type=f32)
    parts = [_fold_to(ws * u_s, 384)]
    avs = [a_s]
    xvp = jnp.dot(xvf, q3_ref[...],
                  preferred_element_type=f32)
    for j in range(3):
        a_vj = jnp.concatenate(
            [xs * shv[:, j:j + 1],
             xvp[:, j * MUL_V:(j + 1) * MUL_V] * shs],
            axis=1).astype(bf16)
        avs.append(a_vj)
        u_vj = jnp.dot(a_vj, rv_ref[...], preferred_element_type=f32)
        parts.append(_fold_to(wv * u_vj, 128))
    big = jnp.concatenate(parts, axis=1).astype(bf16)
    abig = jnp.concatenate(avs, axis=1)
    out = C_PATH * (jnp.dot(big, sf_ref[...], preferred_element_type=f32)
                    + jnp.dot(abig, bm_ref[...],
                              preferred_element_type=f32))
    n = out.shape[0]
    pad = jnp.concatenate(
        [jnp.ones((n, 1), jnp.float32), jnp.zeros((n, 31), jnp.float32)],
        axis=1)
    o_ref[...] = jnp.concatenate([out, pad], axis=1)


def _tc_edge(ea, sh, xd, w1, b1, gs, gv, rs, rv, sel, q3, sf, bm):
    n_blk = ea.shape[0] // BE
    full = lambda r, c: pl.BlockSpec((r, c), lambda i: (0, 0))
    out = pl.pallas_call(
        _tc_edge_body,
        grid=(n_blk,),
        in_specs=[
            pl.BlockSpec((BE, N_EDGE_FEAT), lambda i: (i, 0)),
            pl.BlockSpec((BE, SH_DIM), lambda i: (i, 0)),
            pl.BlockSpec((BE, 128), lambda i: (i, 0)),
            full(N_EDGE_FEAT, HIDDEN),
            full(8, HIDDEN),
            full(HIDDEN, 3072),
            full(HIDDEN, 1024),
            full(64, 3072),
            full(64, 1024),
            full(MUL_S, MUL_V),
            full(MUL_S, MUL_S),
            full(768, NODE_DIM),
            full(256, NODE_DIM),
        ],
        out_specs=pl.BlockSpec((BE, 128), lambda i: (i, 0)),
        out_shape=jax.ShapeDtypeStruct((ea.shape[0], 128), jnp.float32),
    )
    return out(ea, sh, xd, w1, b1, gs, gv, rs, rv, sel, q3, sf, bm)


def _tc_ln_body(p0_ref, p1_ref, na_ref, lnc_ref, m2_ref, o_ref):
    psum = p0_ref[...] + p1_ref[...]
    ssum = psum[:, :NODE_DIM]
    cnt = psum[:, NODE_DIM:NODE_DIM + 1]
    x = ssum / jnp.maximum(cnt, 1.0) + na_ref[...]
    lnc = lnc_ref[...]
    w_s = lnc[0:1, :]
    b_s = lnc[1:2, :]
    ms_s = lnc[2:3, :]
    w_v = lnc[3:4, :]
    ms_v = lnc[4:5, :]
    f1 = x[:, :MUL_S]
    m1 = jnp.mean(f1, axis=1, keepdims=True)
    f1 = f1 - m1 * ms_s
    n1 = jnp.mean(f1 * f1, axis=1, keepdims=True)
    f1 = f1 * (lax.rsqrt(n1 + EPS) * w_s) + b_s
    x2 = x[:, MUL_S:]
    m2f = jnp.dot(x2, m2_ref[...])
    f2 = x2 - m2f * ms_v
    n2 = jnp.mean(f2 * f2, axis=1, keepdims=True)
    f2 = f2 * (lax.rsqrt(n2 + EPS) * w_v)
    o_ref[...] = jnp.concatenate([f1, f2], axis=1)


def _tc_ln(p0, p1, na_pad, lnc, m2c):
    n_blk = N_ACC // BN
    out = pl.pallas_call(
        _tc_ln_body,
        grid=(n_blk,),
        in_specs=[
            pl.BlockSpec((BN, 128), lambda i: (i, 0)),
            pl.BlockSpec((BN, 128), lambda i: (i, 0)),
            pl.BlockSpec((BN, NODE_DIM), lambda i: (i, 0)),
            pl.BlockSpec((8, MUL_S), lambda i: (0, 0)),
            pl.BlockSpec((MUL_S, MUL_S), lambda i: (0, 0)),
        ],
        out_specs=pl.BlockSpec((BN, NODE_DIM), lambda i: (i, 0)),
        out_shape=jax.ShapeDtypeStruct((N_ACC, NODE_DIM), jnp.float32),
    )
    return out(p0, p1, na_pad, lnc, m2c)


def _pad8(row):
    return np.pad(row[None, :], ((0, 7), (0, 0))).astype(np.float32)


_R_S = np.kron(np.eye(64), np.ones((1, MUL_S))).astype(np.float32)
_S_S = np.kron(np.ones((64, 1)), np.eye(MUL_S)).astype(np.float32)
_R_V = np.kron(np.eye(64), np.ones((1, MUL_V))).astype(np.float32)
_S_V = np.kron(np.ones((64, 1)), np.eye(MUL_V)).astype(np.float32)
_SEL = np.kron(np.eye(MUL_V), np.ones((3, 1))).astype(np.float32)
_Q3 = np.zeros((MUL_S, MUL_S), np.float32)
for _u in range(MUL_V):
    for _j in range(3):
        _Q3[3 * _u + _j, _j * MUL_V + _u] = 1.0
_SF = np.zeros((768, NODE_DIM), np.float32)
_SF[:384, :MUL_S] = np.kron(np.ones((8, 1)), np.eye(MUL_S))
for _j in range(3):
    _pj = np.zeros((MUL_V, MUL_S), np.float32)
    for _w in range(MUL_V):
        _pj[_w, 3 * _w + _j] = 1.0
    _SF[384 + _j * 128:384 + (_j + 1) * 128, MUL_S:] = np.kron(
        np.ones((8, 1)), _pj)
_M2 = np.kron(np.ones((MUL_V, MUL_V), np.float32) / MUL_V,
              np.eye(3, dtype=np.float32))


def kernel(node_attr, edge_index, edge_attr, edge_sh, fc_w1, fc_b1, fc_w2,
           fc_b2, ln_weight, ln_bias, ln_mean_shift):
    f32 = jnp.float32
    edge_dst = jnp.concatenate(
        [edge_index[1], jnp.zeros((E_PAD - N_EDGES,), jnp.int32)])
    edge_src = jnp.concatenate(
        [edge_index[0], jnp.full((E_PAD - N_EDGES,), N_NODES, jnp.int32)])
    ea_pad = jnp.concatenate(
        [edge_attr, jnp.zeros((E_PAD - N_EDGES, N_EDGE_FEAT), f32)])
    sh_pad = jnp.concatenate(
        [edge_sh, jnp.zeros((E_PAD - N_EDGES, SH_DIM), f32)])
    na_pad = jnp.concatenate(
        [node_attr, jnp.zeros((N_ACC - N_NODES, NODE_DIM), f32)])
    na128 = jnp.concatenate(
        [node_attr, jnp.zeros((N_NODES, 128 - NODE_DIM), f32)], axis=1)

    bf16 = jnp.bfloat16
    g_s = jnp.concatenate([fc_w2[:, :2304], fc_w2[:, 3328:]],
                          axis=1).astype(bf16)
    g_v = fc_w2[:, 2304:3328].astype(bf16)
    b1r = jnp.tile(fc_b1[None, :], (8, 1))
    b2s = jnp.concatenate([fc_b2[:2304], fc_b2[3328:]]).reshape(64, MUL_S)
    b2v = fc_b2[2304:3328].reshape(64, MUL_V)
    eye3 = jnp.eye(3, dtype=f32)
    zb = jnp.zeros((64, MUL_S), f32)
    blocks = [jnp.concatenate([b2s, zb], axis=1)]
    for _j in range(3):
        pv = (b2v[:, :, None] * eye3[_j][None, None, :]).reshape(64, MUL_S)
        blocks.append(jnp.concatenate([zb, pv], axis=1))
    bmat = jnp.concatenate(blocks, axis=0).astype(jnp.bfloat16)

    lnc = jnp.zeros((8, MUL_S), f32)
    lnc = lnc.at[0].set(ln_weight[:MUL_S])
    lnc = lnc.at[1].set(ln_bias)
    lnc = lnc.at[2].set(ln_mean_shift[0, :MUL_S, 0])
    lnc = lnc.at[3].set(jnp.repeat(ln_weight[MUL_S:], 3))
    lnc = lnc.at[4].set(jnp.repeat(ln_mean_shift[0, MUL_S:, 0], 3))

    z128 = jnp.zeros((NPW, 128), f32)

    dst3d = edge_dst.reshape(NW, NCH, CHUNK)
    src3d = edge_src.reshape(NW, NCH, CHUNK)
    consts = (fc_w1.astype(bf16), b1r, g_s, g_v,
              jnp.asarray(_R_S, bf16), jnp.asarray(_R_V, bf16),
              jnp.asarray(_SEL), jnp.asarray(_Q3), jnp.asarray(_SF, bf16),
              bmat)
    x_dst = _sc_gather(na128, dst3d, E_PAD)
    tp = _tc_edge(ea_pad.astype(bf16), sh_pad, x_dst, *consts)
    summed = _sc_scatter(tp, src3d, z128, E_PAD).reshape(NC, N_ACC, 128)
    out = _tc_ln(summed[0], summed[1], na_pad, lnc, jnp.asarray(_M2))
    return out[:N_NODES]


# speedup vs baseline: 1.3489x; 1.0196x over previous
import functools

import numpy as np
import jax
import jax.numpy as jnp
from jax import lax
from jax.experimental import pallas as pl
from jax.experimental.pallas import tpu as pltpu
from jax.experimental.pallas import tpu_sc as plsc

N_NODES = 10000
N_EDGES = 40000
MUL_S = 48
MUL_V = 16
NODE_DIM = 96
SH_DIM = 4
N_EDGE_FEAT = 128
HIDDEN = 128
C_PATH = 0.125
EPS = 1e-5

NC = 2
NS = 16
NW = NC * NS
E_PAD = 40960
EPW = E_PAD // NW
CHUNK = 128
NCH = EPW // CHUNK
N_ACC = 10240
NPW = N_ACC // NS

BE = 2048
BN = 512


def _make_gather_body(epw, nch):
    def body(table_hbm, idx_hbm, out_hbm, idx_v, rows_v,
             sem0, sem1, sem2, sem3, wsem0, wsem1):
        wid = lax.axis_index("s") * NC + lax.axis_index("c")
        sems = (sem0, sem1, sem2, sem3)
        wsems = (wsem0, wsem1)
        pltpu.sync_copy(idx_hbm.at[wid], idx_v)
        cps = [None] * nch
        wcps = [None] * nch
        for j in range(min(3, nch)):
            cps[j] = pltpu.async_copy(table_hbm.at[idx_v.at[j]],
                                      rows_v.at[j], sems[j])
        for j in range(nch):
            cps[j].wait()
            if j + 3 < nch:
                if j >= 1 and wcps[j - 1] is not None:
                    wcps[j - 1].wait()
                    wcps[j - 1] = None
                cps[j + 3] = pltpu.async_copy(table_hbm.at[idx_v.at[j + 3]],
                                              rows_v.at[(j + 3) % 4],
                                              sems[(j + 3) % 4])
            wcps[j] = pltpu.async_copy(
                rows_v.at[j % 4],
                out_hbm.at[pl.ds(wid * epw + j * CHUNK, CHUNK)],
                wsems[j % 2])
        for j in range(nch):
            if wcps[j] is not None:
                wcps[j].wait()
    return body


def _sc_gather(table_f32, idx3d, n_edges):
    epw = n_edges // NW
    nch = epw // CHUNK
    fn = pl.kernel(
        _make_gather_body(epw, nch),
        out_type=jax.ShapeDtypeStruct((n_edges, 128), jnp.float32),
        mesh=plsc.VectorSubcoreMesh(
            core_axis_name="c", subcore_axis_name="s", num_cores=NC,
            num_subcores=NS,
        ),
        scratch_types=[
            pltpu.VMEM((nch, CHUNK), jnp.int32),
            pltpu.VMEM((4, CHUNK, 128), jnp.float32),
            pltpu.SemaphoreType.DMA,
            pltpu.SemaphoreType.DMA,
            pltpu.SemaphoreType.DMA,
            pltpu.SemaphoreType.DMA,
            pltpu.SemaphoreType.DMA,
            pltpu.SemaphoreType.DMA,
        ],
    )
    return fn(table_f32, idx3d)


def _make_scatter_body(epw, nch):
    def body(tp_hbm, idx_hbm, z_hbm, out_hbm, idx_v, rows_v, acc_sp, lsem):
        c = lax.axis_index("c")
        s = lax.axis_index("s")
        wid = s * NC + c
        pltpu.sync_copy(z_hbm, acc_sp.at[pl.ds(s * NPW, NPW)])
        pltpu.sync_copy(idx_hbm.at[wid], idx_v)
        loads = [None] * nch
        loads[0] = pltpu.async_copy(tp_hbm.at[pl.ds(wid * epw, CHUNK)],
                                    rows_v.at[0], lsem)
        plsc.subcore_barrier()
        for j in range(nch):
            loads[j].wait()
            if j + 1 < nch:
                loads[j + 1] = pltpu.async_copy(
                    tp_hbm.at[pl.ds(wid * epw + (j + 1) * CHUNK, CHUNK)],
                    rows_v.at[(j + 1) % 2], lsem)
            pltpu.sync_copy(rows_v.at[j % 2], acc_sp.at[idx_v.at[j]],
                            add=True)
        plsc.subcore_barrier()
        base = c * N_ACC + s * NPW
        pltpu.sync_copy(acc_sp.at[pl.ds(s * NPW, NPW)],
                        out_hbm.at[pl.ds(base, NPW)])
    return body


def _sc_scatter(tp, idx3d, z128, n_edges):
    epw = n_edges // NW
    nch = epw // CHUNK
    fn = pl.kernel(
        _make_scatter_body(epw, nch),
        out_type=jax.ShapeDtypeStruct((NC * N_ACC, 128), jnp.float32),
        mesh=plsc.VectorSubcoreMesh(
            core_axis_name="c", subcore_axis_name="s", num_cores=NC,
            num_subcores=NS,
        ),
        scratch_types=[
            pltpu.VMEM((nch, CHUNK), jnp.int32),
            pltpu.VMEM((2, CHUNK, 128), jnp.float32),
            pltpu.VMEM_SHARED((N_ACC, 128), jnp.float32),
            pltpu.SemaphoreType.DMA,
        ],
    )
    return fn(tp, idx3d, z128)


def _fold_to(p, target):
    w = p.shape[1]
    while w > target:
        w //= 2
        p = p[:, :w] + p[:, w:2 * w]
    return p


def _tc_edge_body(ea_ref, sh_ref, xd_ref, w1_ref, b1_ref, gs_ref,
                  gv_ref, rs_ref, rv_ref, sel_ref,
                  q3_ref, sf_ref, bm_ref, o_ref):
    f32 = jnp.float32
    bf16 = jnp.bfloat16
    ea = ea_ref[...]
    sh = sh_ref[...]
    xd = xd_ref[:, :NODE_DIM]
    h = jax.nn.relu(
        jnp.dot(ea, w1_ref[...], preferred_element_type=f32)
        + b1_ref[0:1, :])
    hb = h.astype(bf16)
    ws = jnp.dot(hb, gs_ref[...], preferred_element_type=f32)
    wv = jnp.dot(hb, gv_ref[...], preferred_element_type=f32)
    xs = xd[:, :MUL_S]
    xvf = xd[:, MUL_S:]
    shs = sh[:, 0:1]
    shv = sh[:, 1:4]
    vv = jnp.concatenate([shv] * MUL_V, axis=1)
    bb = jnp.dot(xvf * vv, sel_ref[...])
    a_s = jnp.concatenate([xs * shs, bb], axis=1).astype(bf16)
    u_s = jnp.dot(a_s, rs_ref[...], preferred_element_type=f32)
    parts = [_fold_to(ws * u_s, 384)]
    avs = [a_s]
    xvp = jnp.dot(xvf, q3_ref[...],
                  preferred_element_type=f32)
    for j in range(3):
        a_vj = jnp.concatenate(
            [xs * shv[:, j:j + 1],
             xvp[:, j * MUL_V:(j + 1) * MUL_V] * shs],
            axis=1).astype(bf16)
        avs.append(a_vj)
        u_vj = jnp.dot(a_vj, rv_ref[...], preferred_element_type=f32)
        parts.append(_fold_to(wv * u_vj, 128))
    big = jnp.concatenate(parts, axis=1).astype(bf16)
    abig = jnp.concatenate(avs, axis=1)
    out = C_PATH * (jnp.dot(big, sf_ref[...], preferred_element_type=f32)
                    + jnp.dot(abig, bm_ref[...],
                              preferred_element_type=f32))
    n = out.shape[0]
    pad = jnp.concatenate(
        [jnp.ones((n, 1), jnp.float32), jnp.zeros((n, 31), jnp.float32)],
        axis=1)
    o_ref[...] = jnp.concatenate([out, pad], axis=1)


def _tc_edge(ea, sh, xd, w1, b1, gs, gv, rs, rv, sel, q3, sf, bm):
    n_blk = ea.shape[0] // BE
    full = lambda r, c: pl.BlockSpec((r, c), lambda i: (0, 0))
    out = pl.pallas_call(
        _tc_edge_body,
        grid=(n_blk,),
        in_specs=[
            pl.BlockSpec((BE, N_EDGE_FEAT), lambda i: (i, 0)),
            pl.BlockSpec((BE, SH_DIM), lambda i: (i, 0)),
            pl.BlockSpec((BE, 128), lambda i: (i, 0)),
            full(N_EDGE_FEAT, HIDDEN),
            full(8, HIDDEN),
            full(HIDDEN, 3072),
            full(HIDDEN, 1024),
            full(64, 3072),
            full(64, 1024),
            full(MUL_S, MUL_V),
            full(MUL_S, MUL_S),
            full(768, NODE_DIM),
            full(256, NODE_DIM),
        ],
        out_specs=pl.BlockSpec((BE, 128), lambda i: (i, 0)),
        out_shape=jax.ShapeDtypeStruct((ea.shape[0], 128), jnp.float32),
    )
    return out(ea, sh, xd, w1, b1, gs, gv, rs, rv, sel, q3, sf, bm)


def _tc_ln_body(p0_ref, p1_ref, na_ref, lnc_ref, m2_ref, o_ref):
    psum = p0_ref[...] + p1_ref[...]
    ssum = psum[:, :NODE_DIM]
    cnt = psum[:, NODE_DIM:NODE_DIM + 1]
    x = ssum / jnp.maximum(cnt, 1.0) + na_ref[...]
    lnc = lnc_ref[...]
    w_s = lnc[0:1, :]
    b_s = lnc[1:2, :]
    ms_s = lnc[2:3, :]
    w_v = lnc[3:4, :]
    ms_v = lnc[4:5, :]
    f1 = x[:, :MUL_S]
    m1 = jnp.mean(f1, axis=1, keepdims=True)
    f1 = f1 - m1 * ms_s
    n1 = jnp.mean(f1 * f1, axis=1, keepdims=True)
    f1 = f1 * (lax.rsqrt(n1 + EPS) * w_s) + b_s
    x2 = x[:, MUL_S:]
    m2f = jnp.dot(x2, m2_ref[...])
    f2 = x2 - m2f * ms_v
    n2 = jnp.mean(f2 * f2, axis=1, keepdims=True)
    f2 = f2 * (lax.rsqrt(n2 + EPS) * w_v)
    o_ref[...] = jnp.concatenate([f1, f2], axis=1)


def _tc_ln(p0, p1, na_pad, lnc, m2c):
    n_blk = N_ACC // BN
    out = pl.pallas_call(
        _tc_ln_body,
        grid=(n_blk,),
        in_specs=[
            pl.BlockSpec((BN, 128), lambda i: (i, 0)),
            pl.BlockSpec((BN, 128), lambda i: (i, 0)),
            pl.BlockSpec((BN, NODE_DIM), lambda i: (i, 0)),
            pl.BlockSpec((8, MUL_S), lambda i: (0, 0)),
            pl.BlockSpec((MUL_S, MUL_S), lambda i: (0, 0)),
        ],
        out_specs=pl.BlockSpec((BN, NODE_DIM), lambda i: (i, 0)),
        out_shape=jax.ShapeDtypeStruct((N_ACC, NODE_DIM), jnp.float32),
    )
    return out(p0, p1, na_pad, lnc, m2c)


def _pad8(row):
    return np.pad(row[None, :], ((0, 7), (0, 0))).astype(np.float32)


_R_S = np.kron(np.eye(64), np.ones((1, MUL_S))).astype(np.float32)
_S_S = np.kron(np.ones((64, 1)), np.eye(MUL_S)).astype(np.float32)
_R_V = np.kron(np.eye(64), np.ones((1, MUL_V))).astype(np.float32)
_S_V = np.kron(np.ones((64, 1)), np.eye(MUL_V)).astype(np.float32)
_SEL = np.kron(np.eye(MUL_V), np.ones((3, 1))).astype(np.float32)
_Q3 = np.zeros((MUL_S, MUL_S), np.float32)
for _u in range(MUL_V):
    for _j in range(3):
        _Q3[3 * _u + _j, _j * MUL_V + _u] = 1.0
_SF = np.zeros((768, NODE_DIM), np.float32)
_SF[:384, :MUL_S] = np.kron(np.ones((8, 1)), np.eye(MUL_S))
for _j in range(3):
    _pj = np.zeros((MUL_V, MUL_S), np.float32)
    for _w in range(MUL_V):
        _pj[_w, 3 * _w + _j] = 1.0
    _SF[384 + _j * 128:384 + (_j + 1) * 128, MUL_S:] = np.kron(
        np.ones((8, 1)), _pj)
_M2 = np.kron(np.ones((MUL_V, MUL_V), np.float32) / MUL_V,
              np.eye(3, dtype=np.float32))


def kernel(node_attr, edge_index, edge_attr, edge_sh, fc_w1, fc_b1, fc_w2,
           fc_b2, ln_weight, ln_bias, ln_mean_shift):
    f32 = jnp.float32
    edge_dst = jnp.concatenate(
        [edge_index[1], jnp.zeros((E_PAD - N_EDGES,), jnp.int32)])
    edge_src = jnp.concatenate(
        [edge_index[0], jnp.full((E_PAD - N_EDGES,), N_NODES, jnp.int32)])
    ea_pad = jnp.concatenate(
        [edge_attr, jnp.zeros((E_PAD - N_EDGES, N_EDGE_FEAT), f32)])
    sh_pad = jnp.concatenate(
        [edge_sh, jnp.zeros((E_PAD - N_EDGES, SH_DIM), f32)])
    na_pad = jnp.concatenate(
        [node_attr, jnp.zeros((N_ACC - N_NODES, NODE_DIM), f32)])
    na128 = jnp.concatenate(
        [node_attr, jnp.zeros((N_NODES, 128 - NODE_DIM), f32)], axis=1)

    bf16 = jnp.bfloat16
    g_s = jnp.concatenate([fc_w2[:, :2304], fc_w2[:, 3328:]],
                          axis=1).astype(bf16)
    g_v = fc_w2[:, 2304:3328].astype(bf16)
    b1r = jnp.tile(fc_b1[None, :], (8, 1))
    b2s = jnp.concatenate([fc_b2[:2304], fc_b2[3328:]]).reshape(64, MUL_S)
    b2v = fc_b2[2304:3328].reshape(64, MUL_V)
    eye3 = jnp.eye(3, dtype=f32)
    zb = jnp.zeros((64, MUL_S), f32)
    blocks = [jnp.concatenate([b2s, zb], axis=1)]
    for _j in range(3):
        pv = (b2v[:, :, None] * eye3[_j][None, None, :]).reshape(64, MUL_S)
        blocks.append(jnp.concatenate([zb, pv], axis=1))
    bmat = jnp.concatenate(blocks, axis=0).astype(jnp.bfloat16)

    lnc = jnp.zeros((8, MUL_S), f32)
    lnc = lnc.at[0].set(ln_weight[:MUL_S])
    lnc = lnc.at[1].set(ln_bias)
    lnc = lnc.at[2].set(ln_mean_shift[0, :MUL_S, 0])
    lnc = lnc.at[3].set(jnp.repeat(ln_weight[MUL_S:], 3))
    lnc = lnc.at[4].set(jnp.repeat(ln_mean_shift[0, MUL_S:, 0], 3))

    z128 = jnp.zeros((NPW, 128), f32)

    dst3d = edge_dst.reshape(NW, NCH, CHUNK)
    src3d = edge_src.reshape(NW, NCH, CHUNK)
    consts = (fc_w1.astype(bf16), b1r, g_s, g_v,
              jnp.asarray(_R_S, bf16), jnp.asarray(_R_V, bf16),
              jnp.asarray(_SEL), jnp.asarray(_Q3), jnp.asarray(_SF, bf16),
              bmat)
    x_dst = _sc_gather(na128, dst3d, E_PAD)
    tp = _tc_edge(ea_pad.astype(bf16), sh_pad, x_dst, *consts)
    summed = _sc_scatter(tp, src3d, z128, E_PAD).reshape(NC, N_ACC, 128)
    out = _tc_ln(summed[0], summed[1], na_pad, lnc, jnp.asarray(_M2))
    return out[:N_NODES]
